# baseline (speedup 1.0000x reference)
import jax
import jax.numpy as jnp
from jax.experimental import pallas as pl
from jax.experimental.pallas import tpu as pltpu

N = 5000
D = 12544
H = 1024
BM = 1000
BK = 1792
KBLKS = D // BK


def _body(x_ref, w1_ref, b1_ref, w2_ref, b2_ref, w3_ref, b3_ref,
          w4_ref, b4_ref, cls_ref, box_ref, acc_ref):
    k = pl.program_id(1)

    @pl.when(k == 0)
    def _():
        acc_ref[...] = jnp.zeros_like(acc_ref)

    acc_ref[...] += jnp.dot(x_ref[...], w1_ref[...],
                            preferred_element_type=jnp.float32)

    @pl.when(k == KBLKS - 1)
    def _():
        h1 = jnp.maximum(acc_ref[...] + b1_ref[...], 0.0)
        h2 = jnp.maximum(
            jnp.dot(h1, w2_ref[...], preferred_element_type=jnp.float32)
            + b2_ref[...], 0.0)
        logits = jnp.dot(h2, w3_ref[...],
                         preferred_element_type=jnp.float32) + b3_ref[...]
        m = jnp.max(logits, axis=-1, keepdims=True)
        e = jnp.exp(logits - m)
        cls_ref[...] = e / jnp.sum(e, axis=-1, keepdims=True)
        box_ref[...] = jnp.dot(h2, w4_ref[...],
                               preferred_element_type=jnp.float32) + b4_ref[...]


def kernel(feature_vectors, W1, b1, W2, b2, W3, b3, W4, b4):
    C1 = W3.shape[1]
    C4 = W4.shape[1]
    grid = (N // BM, KBLKS)
    out = pl.pallas_call(
        _body,
        grid=grid,
        in_specs=[
            pl.BlockSpec((BM, BK), lambda i, k: (i, k)),
            pl.BlockSpec((BK, H), lambda i, k: (k, 0)),
            pl.BlockSpec((1, H), lambda i, k: (0, 0)),
            pl.BlockSpec((H, H), lambda i, k: (0, 0)),
            pl.BlockSpec((1, H), lambda i, k: (0, 0)),
            pl.BlockSpec((H, C1), lambda i, k: (0, 0)),
            pl.BlockSpec((1, C1), lambda i, k: (0, 0)),
            pl.BlockSpec((H, C4), lambda i, k: (0, 0)),
            pl.BlockSpec((1, C4), lambda i, k: (0, 0)),
        ],
        out_specs=[
            pl.BlockSpec((BM, C1), lambda i, k: (i, 0)),
            pl.BlockSpec((BM, C4), lambda i, k: (i, 0)),
        ],
        out_shape=[
            jax.ShapeDtypeStruct((N, C1), jnp.float32),
            jax.ShapeDtypeStruct((N, C4), jnp.float32),
        ],
        scratch_shapes=[pltpu.VMEM((BM, H), jnp.float32)],
        compiler_params=pltpu.CompilerParams(
            dimension_semantics=("parallel", "arbitrary"),
        ),
    )(feature_vectors, W1, b1.reshape(1, H), W2, b2.reshape(1, H),
      W3, b3.reshape(1, C1), W4, b4.reshape(1, C4))
    return (out[0], out[1])
